# Initial kernel scaffold; baseline (speedup 1.0000x reference)
import jax
import jax.numpy as jnp
from jax.experimental import pallas as pl


def kernel(inputs):
    raise NotImplementedError("write your pallas kernel here")


# speedup vs baseline: 2.3349x; 2.3349x over previous
from math import comb

import numpy as np
import jax
import jax.numpy as jnp
from jax.experimental import pallas as pl
from jax.experimental.pallas import tpu as pltpu

_H, _W = 256, 256
_S = 32
_K = 4
_WIDTH = 4.0
_AAF = 0.35
_EPS = 1e-6


def _basis_T() -> np.ndarray:
    ts = np.linspace(0.0, 1.0, _S).astype(np.float32)
    out = np.zeros((_K, _S), dtype=np.float32)
    for i in range(_K):
        out[i] = comb(_K - 1, i) * ts**i * (1.0 - ts) ** (_K - 1 - i)
    return out


_BASIS_T = _basis_T()


def _bf16(x):
    return x.astype(jnp.bfloat16).astype(jnp.float32)


def _curve_kernel(kp_ref, basis_ref, out_ref):
    kp = kp_ref[0] * jnp.float32(_H)
    kpb = _bf16(kp)

    sy = jnp.zeros((1, _S), dtype=jnp.float32)
    sx = jnp.zeros((1, _S), dtype=jnp.float32)
    for k in range(_K):
        brow = basis_ref[k : k + 1]
        sy = sy + kpb[k, 0] * brow
        sx = sx + kpb[k, 1] * brow

    arow = jnp.float32(-2.0) * _bf16(sy)
    brow = jnp.float32(-2.0) * _bf16(sx)
    crow = sy * sy + sx * sx

    ygrid = jax.lax.broadcasted_iota(jnp.int32, (_H, _W), 0).astype(jnp.float32)
    xrow = jax.lax.broadcasted_iota(jnp.int32, (1, _W), 1).astype(jnp.float32)

    minu = None
    for s in range(_S):
        a = arow[0, s]
        b = brow[0, s]
        c = crow[0, s]
        row = xrow * b + c
        u = ygrid * a + row
        minu = u if minu is None else jnp.minimum(minu, u)

    x2row = xrow * xrow
    d2 = (ygrid * ygrid + x2row) + minu
    mind = jnp.sqrt(jnp.maximum(d2, 0.0))
    arg = mind * jnp.float32(1.0 / _WIDTH) + jnp.float32(_EPS)
    out_ref[0] = 1.0 - jnp.exp2(jnp.log2(arg) * jnp.float32(_AAF))


def kernel(inputs):
    B = inputs.shape[0]
    basis = jnp.asarray(_BASIS_T).astype(jnp.bfloat16).astype(jnp.float32)
    return pl.pallas_call(
        _curve_kernel,
        grid=(B,),
        in_specs=[
            pl.BlockSpec((1, _K, 2), lambda b: (b, 0, 0)),
            pl.BlockSpec((_K, _S), lambda b: (0, 0)),
        ],
        out_specs=pl.BlockSpec((1, _H, _W), lambda b: (b, 0, 0)),
        out_shape=jax.ShapeDtypeStruct((B, _H, _W), jnp.float32),
        compiler_params=pltpu.CompilerParams(
            dimension_semantics=("parallel",),
        ),
    )(inputs, basis)


# speedup vs baseline: 2.9725x; 1.2731x over previous
from math import comb

import numpy as np
import jax
import jax.numpy as jnp
from jax.experimental import pallas as pl
from jax.experimental.pallas import tpu as pltpu

_H, _W = 256, 256
_S = 32
_K = 4
_WIDTH = 4.0
_AAF = 0.35
_ROWS = 64


def _basis_T() -> np.ndarray:
    ts = np.linspace(0.0, 1.0, _S).astype(np.float32)
    out = np.zeros((_K, _S), dtype=np.float32)
    for i in range(_K):
        out[i] = comb(_K - 1, i) * ts**i * (1.0 - ts) ** (_K - 1 - i)
    return out


_BASIS_T = _basis_T()


def _bf16(x):
    return x.astype(jnp.bfloat16).astype(jnp.float32)


def _curve_kernel(kp_ref, basis_ref, out_ref):
    kp = kp_ref[0] * jnp.float32(_H)
    kpb = _bf16(kp)

    sy = jnp.zeros((1, _S), dtype=jnp.float32)
    sx = jnp.zeros((1, _S), dtype=jnp.float32)
    for k in range(_K):
        brow = basis_ref[k : k + 1]
        sy = sy + kpb[k, 0] * brow
        sx = sx + kpb[k, 1] * brow

    arow = jnp.float32(-2.0) * _bf16(sy)
    brow = jnp.float32(-2.0) * _bf16(sx)
    crow = sy * sy + sx * sx

    xrow = jax.lax.broadcasted_iota(jnp.int32, (1, _W), 1).astype(jnp.float32)
    x2row = xrow * xrow

    a = [arow[0, s] for s in range(_S)]
    rows = [xrow * brow[0, s] + crow[0, s] for s in range(_S)]

    yiota = jax.lax.broadcasted_iota(jnp.int32, (_ROWS, _W), 0).astype(
        jnp.float32
    )
    c0 = jnp.float32(_AAF * 0.5)
    c1 = jnp.float32(_AAF * -2.0)
    for r in range(_H // _ROWS):
        yb = yiota + jnp.float32(r * _ROWS)
        minu = yb * a[0] + rows[0]
        for s in range(1, _S):
            minu = jnp.minimum(minu, yb * a[s] + rows[s])
        d2 = (yb * yb + x2row) + minu
        d2 = jnp.maximum(d2, 0.0)
        out_ref[0, r * _ROWS : (r + 1) * _ROWS, :] = 1.0 - jnp.exp2(
            jnp.log2(d2) * c0 + c1
        )


def kernel(inputs):
    B = inputs.shape[0]
    basis = jnp.asarray(_BASIS_T).astype(jnp.bfloat16).astype(jnp.float32)
    return pl.pallas_call(
        _curve_kernel,
        grid=(B,),
        in_specs=[
            pl.BlockSpec((1, _K, 2), lambda b: (b, 0, 0)),
            pl.BlockSpec((_K, _S), lambda b: (0, 0)),
        ],
        out_specs=pl.BlockSpec((1, _H, _W), lambda b: (b, 0, 0)),
        out_shape=jax.ShapeDtypeStruct((B, _H, _W), jnp.float32),
        compiler_params=pltpu.CompilerParams(
            dimension_semantics=("arbitrary",),
        ),
    )(inputs, basis)


# speedup vs baseline: 3.3672x; 1.1328x over previous
from math import comb

import numpy as np
import jax
import jax.numpy as jnp
from jax.experimental import pallas as pl
from jax.experimental.pallas import tpu as pltpu

_H, _W = 256, 256
_S = 32
_K = 4
_WIDTH = 4.0
_AAF = 0.35
_ROWS = 64
_G = 8


def _basis_T() -> np.ndarray:
    ts = np.linspace(0.0, 1.0, _S).astype(np.float32)
    out = np.zeros((_K, _S), dtype=np.float32)
    for i in range(_K):
        out[i] = comb(_K - 1, i) * ts**i * (1.0 - ts) ** (_K - 1 - i)
    return out


_BASIS_T = _basis_T()

import ml_dtypes

_BASIS_BF16 = _BASIS_T.astype(ml_dtypes.bfloat16).astype(np.float32)

_YS = np.arange(_H, dtype=np.float32)[:, None]
_XS = np.arange(_W, dtype=np.float32)[None, :]
_P2 = _YS * _YS + _XS * _XS


def _bf16(x):
    return x.astype(jnp.bfloat16).astype(jnp.float32)


def _curve_kernel(kp_ref, basis_ref, p2_ref, out_ref):
    xrow = jax.lax.broadcasted_iota(jnp.int32, (1, _W), 1).astype(jnp.float32)
    yiota = jax.lax.broadcasted_iota(jnp.int32, (_ROWS, _W), 0).astype(
        jnp.float32
    )
    c0 = jnp.float32(_AAF * 0.5)
    c1 = jnp.float32(_AAF * -2.0)

    for g in range(_G):
        kp = kp_ref[g] * jnp.float32(_H)
        kpb = _bf16(kp)

        sy = jnp.zeros((1, _S), dtype=jnp.float32)
        sx = jnp.zeros((1, _S), dtype=jnp.float32)
        for k in range(_K):
            brow = basis_ref[k : k + 1]
            sy = sy + kpb[k : k + 1, 0:1] * brow
            sx = sx + kpb[k : k + 1, 1:2] * brow

        arow = jnp.float32(-2.0) * _bf16(sy)
        brow = jnp.float32(-2.0) * _bf16(sx)
        crow = sy * sy + sx * sx

        aa = [arow[0:1, s : s + 1] for s in range(_S)]
        rows = [
            xrow * brow[0:1, s : s + 1] + crow[0:1, s : s + 1]
            for s in range(_S)
        ]

        for r in range(_H // _ROWS):
            yb = yiota + jnp.float32(r * _ROWS)
            minu = yb * aa[0] + rows[0]
            for s in range(1, _S):
                minu = jnp.minimum(minu, yb * aa[s] + rows[s])
            d2 = p2_ref[r * _ROWS : (r + 1) * _ROWS, :] + minu
            d2 = jnp.maximum(d2, 0.0)
            out_ref[g, r * _ROWS : (r + 1) * _ROWS, :] = 1.0 - jnp.exp2(
                jnp.log2(d2) * c0 + c1
            )


def kernel(inputs):
    B = inputs.shape[0]
    basis = jnp.asarray(_BASIS_BF16)
    p2 = jnp.asarray(_P2)
    return pl.pallas_call(
        _curve_kernel,
        grid=(B // _G,),
        in_specs=[
            pl.BlockSpec((_G, _K, 2), lambda b: (b, 0, 0)),
            pl.BlockSpec((_K, _S), lambda b: (0, 0)),
            pl.BlockSpec((_H, _W), lambda b: (0, 0)),
        ],
        out_specs=pl.BlockSpec((_G, _H, _W), lambda b: (b, 0, 0)),
        out_shape=jax.ShapeDtypeStruct((B, _H, _W), jnp.float32),
        compiler_params=pltpu.CompilerParams(
            dimension_semantics=("arbitrary",),
        ),
    )(inputs, basis, p2)


# speedup vs baseline: 3.4595x; 1.0274x over previous
from math import comb

import numpy as np
import jax
import jax.numpy as jnp
from jax.experimental import pallas as pl
from jax.experimental.pallas import tpu as pltpu

_H, _W = 256, 256
_S = 32
_K = 4
_WIDTH = 4.0
_AAF = 0.35
_ROWS = 32
_G = 4


def _basis_T() -> np.ndarray:
    ts = np.linspace(0.0, 1.0, _S).astype(np.float32)
    out = np.zeros((_K, _S), dtype=np.float32)
    for i in range(_K):
        out[i] = comb(_K - 1, i) * ts**i * (1.0 - ts) ** (_K - 1 - i)
    return out


_BASIS_T = _basis_T()

import ml_dtypes

_BASIS_BF16 = _BASIS_T.astype(ml_dtypes.bfloat16).astype(np.float32)

_YS = np.arange(_H, dtype=np.float32)[:, None]
_XS = np.arange(_W, dtype=np.float32)[None, :]
_P2 = _YS * _YS + _XS * _XS


def _bf16(x):
    return x.astype(jnp.bfloat16).astype(jnp.float32)


def _curve_kernel(kp_ref, basis_ref, p2_ref, out_ref):
    xrow = jax.lax.broadcasted_iota(jnp.int32, (1, _W), 1).astype(jnp.float32)
    yiota = jax.lax.broadcasted_iota(jnp.int32, (_ROWS, _W), 0).astype(
        jnp.float32
    )
    c0 = jnp.float32(_AAF * 0.5)
    c1 = jnp.float32(_AAF * -2.0)

    for g in range(_G):
        kp = kp_ref[g] * jnp.float32(_H)
        kpb = _bf16(kp)

        sy = jnp.zeros((1, _S), dtype=jnp.float32)
        sx = jnp.zeros((1, _S), dtype=jnp.float32)
        for k in range(_K):
            brow = basis_ref[k : k + 1]
            sy = sy + kpb[k : k + 1, 0:1] * brow
            sx = sx + kpb[k : k + 1, 1:2] * brow

        arow = jnp.float32(-2.0) * _bf16(sy)
        brow = jnp.float32(-2.0) * _bf16(sx)
        crow = sy * sy + sx * sx

        aa = [arow[0:1, s : s + 1] for s in range(_S)]
        rows = [
            xrow * brow[0:1, s : s + 1] + crow[0:1, s : s + 1]
            for s in range(_S)
        ]

        for r in range(_H // _ROWS):
            yb = yiota + jnp.float32(r * _ROWS)
            minu = yb * aa[0] + rows[0]
            for s in range(1, _S):
                minu = jnp.minimum(minu, yb * aa[s] + rows[s])
            d2 = p2_ref[r * _ROWS : (r + 1) * _ROWS, :] + minu
            d2 = jnp.maximum(d2, 0.0)
            out_ref[g, r * _ROWS : (r + 1) * _ROWS, :] = 1.0 - jnp.exp2(
                jnp.log2(d2) * c0 + c1
            )


def kernel(inputs):
    B = inputs.shape[0]
    basis = jnp.asarray(_BASIS_BF16)
    p2 = jnp.asarray(_P2)
    return pl.pallas_call(
        _curve_kernel,
        grid=(B // _G,),
        in_specs=[
            pl.BlockSpec((_G, _K, 2), lambda b: (b, 0, 0)),
            pl.BlockSpec((_K, _S), lambda b: (0, 0)),
            pl.BlockSpec((_H, _W), lambda b: (0, 0)),
        ],
        out_specs=pl.BlockSpec((_G, _H, _W), lambda b: (b, 0, 0)),
        out_shape=jax.ShapeDtypeStruct((B, _H, _W), jnp.float32),
        compiler_params=pltpu.CompilerParams(
            dimension_semantics=("arbitrary",),
        ),
    )(inputs, basis, p2)


# speedup vs baseline: 3.7693x; 1.0895x over previous
from math import comb

import numpy as np
import jax
import jax.numpy as jnp
from jax.experimental import pallas as pl
from jax.experimental.pallas import tpu as pltpu

_H, _W = 256, 256
_S = 32
_K = 4
_WIDTH = 4.0
_AAF = 0.35
_ROWS = 32
_G = 4


def _basis_T() -> np.ndarray:
    ts = np.linspace(0.0, 1.0, _S).astype(np.float32)
    out = np.zeros((_K, _S), dtype=np.float32)
    for i in range(_K):
        out[i] = comb(_K - 1, i) * ts**i * (1.0 - ts) ** (_K - 1 - i)
    return out


_BASIS_T = _basis_T()

import ml_dtypes

_BASIS_BF16 = np.ascontiguousarray(
    _BASIS_T.astype(ml_dtypes.bfloat16).astype(np.float32).T
)

_YS = np.arange(_H, dtype=np.float32)[:, None]
_XS = np.arange(_W, dtype=np.float32)[None, :]
_P2 = _YS * _YS + _XS * _XS


def _bf16(x):
    return x.astype(jnp.bfloat16).astype(jnp.float32)


def _curve_kernel(kp_ref, basis_ref, p2_ref, out_ref):
    xrow = jax.lax.broadcasted_iota(jnp.int32, (1, _W), 1).astype(jnp.float32)
    yiota = jax.lax.broadcasted_iota(jnp.int32, (_ROWS, _W), 0).astype(
        jnp.float32
    )
    c0 = jnp.float32(_AAF * 0.5)
    c1 = jnp.float32(_AAF * -2.0)

    for g in range(_G):
        kp = kp_ref[g] * jnp.float32(_H)
        kpb = _bf16(kp)

        sy = jnp.zeros((_S, 1), dtype=jnp.float32)
        sx = jnp.zeros((_S, 1), dtype=jnp.float32)
        for k in range(_K):
            bcol = basis_ref[:, k : k + 1]
            sy = sy + kpb[k : k + 1, 0:1] * bcol
            sx = sx + kpb[k : k + 1, 1:2] * bcol

        acol = jnp.float32(-2.0) * _bf16(sy)
        bcol = jnp.float32(-2.0) * _bf16(sx)
        ccol = sy * sy + sx * sx

        aa = [acol[s : s + 1, 0:1] for s in range(_S)]
        rows = [
            xrow * bcol[s : s + 1, 0:1] + ccol[s : s + 1, 0:1]
            for s in range(_S)
        ]

        for r in range(_H // _ROWS):
            yb = yiota + jnp.float32(r * _ROWS)
            minu = yb * aa[0] + rows[0]
            for s in range(1, _S):
                minu = jnp.minimum(minu, yb * aa[s] + rows[s])
            d2 = p2_ref[r * _ROWS : (r + 1) * _ROWS, :] + minu
            d2 = jnp.maximum(d2, 0.0)
            out_ref[g, r * _ROWS : (r + 1) * _ROWS, :] = 1.0 - jnp.exp2(
                jnp.log2(d2) * c0 + c1
            )


def kernel(inputs):
    B = inputs.shape[0]
    basis = jnp.asarray(_BASIS_BF16)
    p2 = jnp.asarray(_P2)
    return pl.pallas_call(
        _curve_kernel,
        grid=(B // _G,),
        in_specs=[
            pl.BlockSpec((_G, _K, 2), lambda b: (b, 0, 0)),
            pl.BlockSpec((_S, _K), lambda b: (0, 0)),
            pl.BlockSpec((_H, _W), lambda b: (0, 0)),
        ],
        out_specs=pl.BlockSpec((_G, _H, _W), lambda b: (b, 0, 0)),
        out_shape=jax.ShapeDtypeStruct((B, _H, _W), jnp.float32),
        compiler_params=pltpu.CompilerParams(
            dimension_semantics=("arbitrary",),
        ),
    )(inputs, basis, p2)
